# Initial kernel scaffold; baseline (speedup 1.0000x reference)
import jax
import jax.numpy as jnp
from jax.experimental import pallas as pl


def kernel(h, batch, W1, b1, W2, b2):
    raise NotImplementedError("write your pallas kernel here")


# speedup vs baseline: 5.9307x; 5.9307x over previous
import functools

import jax
import jax.numpy as jnp
from jax import lax
from jax.experimental import pallas as pl
from jax.experimental.pallas import tpu as pltpu
from jax.experimental.pallas import tpu_sc as plsc

_N = 100000
_H = 128
_G = 512

_BLK = 512

_NC = 2
_NS = 16
_NW = _NC * _NS
_CH = 25
_RW = 128
_NP = _NW * _CH * _RW
_G1 = _G + 8


def _score_body(h_ref, w1_ref, b1_ref, w2_ref, w_ref, er_ref):
    h = h_ref[...]
    t = jnp.tanh(
        jnp.dot(h, w1_ref[...], preferred_element_type=jnp.float32)
        + b1_ref[...]
    )
    w2 = w2_ref[...]
    s = lax.dot_general(
        t, w2, (((1,), (1,)), ((), ())),
        preferred_element_type=jnp.float32,
    )
    e = jnp.exp(s - jnp.sum(jnp.abs(w2)))
    rid = (pl.program_id(0) * _BLK
           + lax.broadcasted_iota(jnp.int32, (e.shape[0], 1), 0))
    e = jnp.where(rid < _N, e, 0.0)
    w_ref[...] = e * h
    er_ref[...] = jnp.broadcast_to(e, (e.shape[0], 16))


def _scatter_body(w_hbm, e_hbm, idx_hbm, outp, outd,
                  idx_cur, wbuf, ebuf, zbuf, dloc, tbufd, drow, accp, accd):
    cid = lax.axis_index("c")
    sid = lax.axis_index("s")
    wid = sid * _NC + cid

    zero = jnp.zeros((16,), jnp.float32)
    for r in range(32):
        for g in range(8):
            zbuf[r, pl.ds(g * 16, 16)] = zero
    for g in range(2):
        dloc[pl.ds(g * 16, 16)] = zero

    def zrow(k, c):
        pltpu.sync_copy(zbuf, accp.at[pl.ds(sid * _G1 + k * 32, 32)])
        pltpu.sync_copy(dloc, accd.at[pl.ds(sid * _G1 + k * 32, 32)])
        return c
    lax.fori_loop(0, 16, zrow, 0)
    plsc.subcore_barrier()

    def chunk(j, carry):
        base = (wid * _CH + j) * _RW
        pltpu.sync_copy(w_hbm.at[pl.ds(base, _RW)], wbuf)
        pltpu.sync_copy(e_hbm.at[pl.ds(base, _RW)], ebuf)
        pltpu.sync_copy(idx_hbm.at[pl.ds(base, _RW)], idx_cur)
        off = sid * _G1
        for g in range(8):
            idx_cur[pl.ds(g * 16, 16)] = idx_cur[pl.ds(g * 16, 16)] + off
        pltpu.sync_copy(wbuf, accp.at[idx_cur], add=True)
        pltpu.sync_copy(ebuf, accd.at[idx_cur], add=True)
        return carry

    lax.fori_loop(0, _CH, chunk, 0)
    plsc.subcore_barrier()

    def prow(k, c):
        pltpu.sync_copy(accp.at[pl.ds(sid * _G1 + k * 32, 32)],
                        outp.at[cid, sid, pl.ds(k * 32, 32)])
        return c
    lax.fori_loop(0, 16, prow, 0)

    for g in range(2):
        dloc[pl.ds(g * 16, 16)] = zero

    def dred(t, c):
        pltpu.sync_copy(accd.at[pl.ds(t * _G1 + sid * 32, 32)], tbufd)
        for g in range(2):
            dloc[pl.ds(g * 16, 16)] = (dloc[pl.ds(g * 16, 16)]
                                       + tbufd[pl.ds(g * 16, 16)])
        return c
    lax.fori_loop(0, 16, dred, 0)
    for g2 in range(2):
        v = dloc[pl.ds(g2 * 16, 16)]
        for k in range(16):
            row = jnp.full((16,), v[k], jnp.float32)
            for g in range(8):
                drow[g2 * 16 + k, pl.ds(g * 16, 16)] = row
    pltpu.sync_copy(drow, outd.at[cid, pl.ds(sid * 32, 32)])


def _combine_body(p_ref, d_ref, o_ref):
    p = p_ref[...]
    d3 = d_ref[...]
    ps = jnp.sum(p, axis=(0, 1))
    d = d3[0, :, 0:1] + d3[1, :, 0:1]
    o_ref[...] = ps / jnp.where(d > 0.0, d, 1.0)


def kernel(h, batch, W1, b1, W2, b2):
    del b2
    w2r = W2.reshape(1, _H)
    b1r = b1.reshape(1, _H)

    grid = (_N + _BLK - 1) // _BLK
    w, er = pl.pallas_call(
        _score_body,
        grid=(grid,),
        in_specs=[
            pl.BlockSpec((_BLK, _H), lambda i: (i, 0)),
            pl.BlockSpec((_H, _H), lambda i: (0, 0)),
            pl.BlockSpec((1, _H), lambda i: (0, 0)),
            pl.BlockSpec((1, _H), lambda i: (0, 0)),
        ],
        out_specs=[
            pl.BlockSpec((_BLK, _H), lambda i: (i, 0)),
            pl.BlockSpec((_BLK, 16), lambda i: (i, 0)),
        ],
        out_shape=[
            jax.ShapeDtypeStruct((_NP, _H), jnp.float32),
            jax.ShapeDtypeStruct((_NP, 16), jnp.float32),
        ],
    )(h, W1, b1r, w2r)

    pad_ids = jnp.full((_NP - _N,), _G, jnp.int32)
    idx_flat = jnp.concatenate([batch.astype(jnp.int32), pad_ids])

    scatter = pl.kernel(
        _scatter_body,
        out_type=[
            jax.ShapeDtypeStruct((_NC, _NS, _G, _H), jnp.float32),
            jax.ShapeDtypeStruct((_NC, _G, _H), jnp.float32),
        ],
        mesh=plsc.VectorSubcoreMesh(
            core_axis_name="c", subcore_axis_name="s",
            num_cores=_NC, num_subcores=_NS,
        ),
        scratch_types=[
            pltpu.VMEM((_RW,), jnp.int32),
            pltpu.VMEM((_RW, _H), jnp.float32),
            pltpu.VMEM((_RW,), jnp.float32),
            pltpu.VMEM((32, _H), jnp.float32),
            pltpu.VMEM((32,), jnp.float32),
            pltpu.VMEM((32,), jnp.float32),
            pltpu.VMEM((32, _H), jnp.float32),
            pltpu.VMEM_SHARED((_NS * _G1, _H), jnp.float32),
            pltpu.VMEM_SHARED((_NS * _G1,), jnp.float32),
        ],
    )
    p, dd = scatter(w, er[:, 0], idx_flat)

    out = pl.pallas_call(
        _combine_body,
        out_shape=jax.ShapeDtypeStruct((_G, _H), jnp.float32),
    )(p, dd)
    return out


# speedup vs baseline: 8.3866x; 1.4141x over previous
import functools

import jax
import jax.numpy as jnp
from jax import lax
from jax.experimental import pallas as pl
from jax.experimental.pallas import tpu as pltpu
from jax.experimental.pallas import tpu_sc as plsc

_N = 100000
_H = 128
_G = 512

_BLK = 2048

_NC = 2
_NS = 16
_NW = _NC * _NS
_CH = 25
_RW = 128
_NP = _NW * _CH * _RW
_G1 = _G + 8


def _score_body(h_ref, w1_ref, b1_ref, w2_ref, w_ref, er_ref):
    h = h_ref[...]
    t = jnp.tanh(
        jnp.dot(h, w1_ref[...], preferred_element_type=jnp.float32)
        + b1_ref[...]
    )
    w2 = w2_ref[...]
    s = lax.dot_general(
        t, w2, (((1,), (1,)), ((), ())),
        preferred_element_type=jnp.float32,
    )
    e = jnp.exp(s - jnp.sum(jnp.abs(w2)))
    w_ref[...] = e * h
    er_ref[...] = jnp.broadcast_to(e, (e.shape[0], 16))


def _scatter_body(w_hbm, e_hbm, idx_hbm, outp, outd,
                  idx_cur, wbuf, ebuf, zbuf, dloc, tbufd, drow, accp, accd):
    cid = lax.axis_index("c")
    sid = lax.axis_index("s")
    wid = sid * _NC + cid

    zero = jnp.zeros((16,), jnp.float32)
    for r in range(32):
        for g in range(8):
            zbuf[r, pl.ds(g * 16, 16)] = zero
    for g in range(2):
        dloc[pl.ds(g * 16, 16)] = zero

    def zrow(k, c):
        pltpu.sync_copy(zbuf, accp.at[pl.ds(sid * _G1 + k * 32, 32)])
        pltpu.sync_copy(dloc, accd.at[pl.ds(sid * _G1 + k * 32, 32)])
        return c
    lax.fori_loop(0, 16, zrow, 0)
    plsc.subcore_barrier()

    def chunk(j, carry):
        base = (wid * _CH + j) * _RW
        pltpu.sync_copy(w_hbm.at[pl.ds(base, _RW)], wbuf)
        pltpu.sync_copy(e_hbm.at[pl.ds(base, _RW)], ebuf)
        pltpu.sync_copy(idx_hbm.at[pl.ds(base, _RW)], idx_cur)
        off = sid * _G1
        for g in range(8):
            idx_cur[pl.ds(g * 16, 16)] = idx_cur[pl.ds(g * 16, 16)] + off
        pltpu.sync_copy(wbuf, accp.at[idx_cur], add=True)
        pltpu.sync_copy(ebuf, accd.at[idx_cur], add=True)
        return carry

    lax.fori_loop(0, _CH, chunk, 0)
    plsc.subcore_barrier()

    def prow(k, c):
        pltpu.sync_copy(accp.at[pl.ds(sid * _G1 + k * 32, 32)],
                        outp.at[cid, sid, pl.ds(k * 32, 32)])
        return c
    lax.fori_loop(0, 16, prow, 0)

    for g in range(2):
        dloc[pl.ds(g * 16, 16)] = zero

    def dred(t, c):
        pltpu.sync_copy(accd.at[pl.ds(t * _G1 + sid * 32, 32)], tbufd)
        for g in range(2):
            dloc[pl.ds(g * 16, 16)] = (dloc[pl.ds(g * 16, 16)]
                                       + tbufd[pl.ds(g * 16, 16)])
        return c
    lax.fori_loop(0, 16, dred, 0)
    for g2 in range(2):
        v = dloc[pl.ds(g2 * 16, 16)]
        for k in range(16):
            row = jnp.full((16,), v[k], jnp.float32)
            for g in range(8):
                drow[g2 * 16 + k, pl.ds(g * 16, 16)] = row
    pltpu.sync_copy(drow, outd.at[cid, pl.ds(sid * 32, 32)])


def _combine_body(p_ref, d_ref, o_ref):
    p = p_ref[...]
    d3 = d_ref[...]
    ps = jnp.sum(p, axis=(0, 1))
    d = d3[0, :, 0:1] + d3[1, :, 0:1]
    o_ref[...] = ps / jnp.where(d > 0.0, d, 1.0)


def kernel(h, batch, W1, b1, W2, b2):
    del b2
    w2r = W2.reshape(1, _H)
    b1r = b1.reshape(1, _H)

    grid = (_N + _BLK - 1) // _BLK
    w, er = pl.pallas_call(
        _score_body,
        grid=(grid,),
        in_specs=[
            pl.BlockSpec((_BLK, _H), lambda i: (i, 0)),
            pl.BlockSpec((_H, _H), lambda i: (0, 0)),
            pl.BlockSpec((1, _H), lambda i: (0, 0)),
            pl.BlockSpec((1, _H), lambda i: (0, 0)),
        ],
        out_specs=[
            pl.BlockSpec((_BLK, _H), lambda i: (i, 0)),
            pl.BlockSpec((_BLK, 16), lambda i: (i, 0)),
        ],
        out_shape=[
            jax.ShapeDtypeStruct((_NP, _H), jnp.float32),
            jax.ShapeDtypeStruct((_NP, 16), jnp.float32),
        ],
    )(h, W1, b1r, w2r)

    pad_ids = jnp.full((_NP - _N,), _G, jnp.int32)
    idx_flat = jnp.concatenate([batch.astype(jnp.int32), pad_ids])

    scatter = pl.kernel(
        _scatter_body,
        out_type=[
            jax.ShapeDtypeStruct((_NC, _NS, _G, _H), jnp.float32),
            jax.ShapeDtypeStruct((_NC, _G, _H), jnp.float32),
        ],
        mesh=plsc.VectorSubcoreMesh(
            core_axis_name="c", subcore_axis_name="s",
            num_cores=_NC, num_subcores=_NS,
        ),
        scratch_types=[
            pltpu.VMEM((_RW,), jnp.int32),
            pltpu.VMEM((_RW, _H), jnp.float32),
            pltpu.VMEM((_RW,), jnp.float32),
            pltpu.VMEM((32, _H), jnp.float32),
            pltpu.VMEM((32,), jnp.float32),
            pltpu.VMEM((32,), jnp.float32),
            pltpu.VMEM((32, _H), jnp.float32),
            pltpu.VMEM_SHARED((_NS * _G1, _H), jnp.float32),
            pltpu.VMEM_SHARED((_NS * _G1,), jnp.float32),
        ],
    )
    p, dd = scatter(w, er[:, 0], idx_flat)

    out = pl.pallas_call(
        _combine_body,
        out_shape=jax.ShapeDtypeStruct((_G, _H), jnp.float32),
    )(p, dd)
    return out


# speedup vs baseline: 8.5565x; 1.0203x over previous
import functools

import jax
import jax.numpy as jnp
from jax import lax
from jax.experimental import pallas as pl
from jax.experimental.pallas import tpu as pltpu
from jax.experimental.pallas import tpu_sc as plsc

_N = 100000
_H = 128
_G = 512

_BLK = 2048

_NC = 2
_NS = 16
_NW = _NC * _NS
_CH = 25
_RW = 128
_NP = _NW * _CH * _RW
_G1 = _G + 8


def _score_body(h_ref, w1_ref, b1_ref, w2_ref, w_ref, er_ref):
    h = h_ref[...]
    t = jnp.tanh(
        jnp.dot(h, w1_ref[...], preferred_element_type=jnp.float32)
        + b1_ref[...]
    )
    w2 = w2_ref[...]
    s = lax.dot_general(
        t, w2, (((1,), (1,)), ((), ())),
        preferred_element_type=jnp.float32,
    )
    e = jnp.exp(s - jnp.sum(jnp.abs(w2)))
    w_ref[...] = e * h
    er_ref[...] = jnp.broadcast_to(e, (e.shape[0], 16))


def _scatter_body(w_hbm, e_hbm, idx_hbm, outp, outd,
                  idx_cur, wbuf, ebuf, zbuf, tbufp, dloc, tbufd, drow,
                  accp, accd):
    cid = lax.axis_index("c")
    sid = lax.axis_index("s")
    wid = sid * _NC + cid

    zero = jnp.zeros((16,), jnp.float32)
    for r in range(32):
        for g in range(8):
            zbuf[r, pl.ds(g * 16, 16)] = zero
    for g in range(2):
        dloc[pl.ds(g * 16, 16)] = zero

    def zrow(k, c):
        pltpu.sync_copy(zbuf, accp.at[pl.ds(sid * _G1 + k * 32, 32)])
        pltpu.sync_copy(dloc, accd.at[pl.ds(sid * _G1 + k * 32, 32)])
        return c
    lax.fori_loop(0, 16, zrow, 0)
    plsc.subcore_barrier()

    def chunk(j, carry):
        base = (wid * _CH + j) * _RW
        pltpu.sync_copy(w_hbm.at[pl.ds(base, _RW)], wbuf)
        pltpu.sync_copy(e_hbm.at[pl.ds(base, _RW)], ebuf)
        pltpu.sync_copy(idx_hbm.at[pl.ds(base, _RW)], idx_cur)
        off = sid * _G1
        for g in range(8):
            idx_cur[pl.ds(g * 16, 16)] = idx_cur[pl.ds(g * 16, 16)] + off
        pltpu.sync_copy(wbuf, accp.at[idx_cur], add=True)
        pltpu.sync_copy(ebuf, accd.at[idx_cur], add=True)
        return carry

    lax.fori_loop(0, _CH, chunk, 0)
    plsc.subcore_barrier()

    def pred(t, c):
        pltpu.sync_copy(accp.at[pl.ds(t * _G1 + sid * 32, 32)], tbufp)

        def prow(r, c2):
            for g in range(8):
                zbuf[r, pl.ds(g * 16, 16)] = (zbuf[r, pl.ds(g * 16, 16)]
                                              + tbufp[r, pl.ds(g * 16, 16)])
            return c2
        lax.fori_loop(0, 32, prow, 0)
        return c
    lax.fori_loop(0, 16, pred, 0)
    pltpu.sync_copy(zbuf, outp.at[cid, pl.ds(sid * 32, 32)])

    for g in range(2):
        dloc[pl.ds(g * 16, 16)] = zero

    def dred(t, c):
        pltpu.sync_copy(accd.at[pl.ds(t * _G1 + sid * 32, 32)], tbufd)
        for g in range(2):
            dloc[pl.ds(g * 16, 16)] = (dloc[pl.ds(g * 16, 16)]
                                       + tbufd[pl.ds(g * 16, 16)])
        return c
    lax.fori_loop(0, 16, dred, 0)
    for g2 in range(2):
        v = dloc[pl.ds(g2 * 16, 16)]
        for k in range(16):
            row = jnp.full((16,), v[k], jnp.float32)
            for g in range(8):
                drow[g2 * 16 + k, pl.ds(g * 16, 16)] = row
    pltpu.sync_copy(drow, outd.at[cid, pl.ds(sid * 32, 32)])


def _combine_body(p_ref, d_ref, o_ref):
    p = p_ref[...]
    d3 = d_ref[...]
    ps = p[0] + p[1]
    d = d3[0, :, 0:1] + d3[1, :, 0:1]
    o_ref[...] = ps / jnp.where(d > 0.0, d, 1.0)


def kernel(h, batch, W1, b1, W2, b2):
    del b2
    w2r = W2.reshape(1, _H)
    b1r = b1.reshape(1, _H)

    grid = (_N + _BLK - 1) // _BLK
    w, er = pl.pallas_call(
        _score_body,
        grid=(grid,),
        in_specs=[
            pl.BlockSpec((_BLK, _H), lambda i: (i, 0)),
            pl.BlockSpec((_H, _H), lambda i: (0, 0)),
            pl.BlockSpec((1, _H), lambda i: (0, 0)),
            pl.BlockSpec((1, _H), lambda i: (0, 0)),
        ],
        out_specs=[
            pl.BlockSpec((_BLK, _H), lambda i: (i, 0)),
            pl.BlockSpec((_BLK, 16), lambda i: (i, 0)),
        ],
        out_shape=[
            jax.ShapeDtypeStruct((_NP, _H), jnp.float32),
            jax.ShapeDtypeStruct((_NP, 16), jnp.float32),
        ],
    )(h, W1, b1r, w2r)

    pad_ids = jnp.full((_NP - _N,), _G, jnp.int32)
    idx_flat = jnp.concatenate([batch.astype(jnp.int32), pad_ids])

    scatter = pl.kernel(
        _scatter_body,
        out_type=[
            jax.ShapeDtypeStruct((_NC, _G, _H), jnp.float32),
            jax.ShapeDtypeStruct((_NC, _G, _H), jnp.float32),
        ],
        mesh=plsc.VectorSubcoreMesh(
            core_axis_name="c", subcore_axis_name="s",
            num_cores=_NC, num_subcores=_NS,
        ),
        scratch_types=[
            pltpu.VMEM((_RW,), jnp.int32),
            pltpu.VMEM((_RW, _H), jnp.float32),
            pltpu.VMEM((_RW,), jnp.float32),
            pltpu.VMEM((32, _H), jnp.float32),
            pltpu.VMEM((32, _H), jnp.float32),
            pltpu.VMEM((32,), jnp.float32),
            pltpu.VMEM((32,), jnp.float32),
            pltpu.VMEM((32, _H), jnp.float32),
            pltpu.VMEM_SHARED((_NS * _G1, _H), jnp.float32),
            pltpu.VMEM_SHARED((_NS * _G1,), jnp.float32),
        ],
    )
    p, dd = scatter(w, er[:, 0], idx_flat)

    out = pl.pallas_call(
        _combine_body,
        out_shape=jax.ShapeDtypeStruct((_G, _H), jnp.float32),
    )(p, dd)
    return out


# speedup vs baseline: 10.8646x; 1.2697x over previous
import functools

import jax
import jax.numpy as jnp
from jax import lax
from jax.experimental import pallas as pl
from jax.experimental.pallas import tpu as pltpu
from jax.experimental.pallas import tpu_sc as plsc

_N = 100000
_H = 128
_G = 512

_BLK = 2048

_NC = 2
_NS = 16
_NW = _NC * _NS
_CH = 25
_RW = 128
_NP = _NW * _CH * _RW
_G1 = _G + 8


def _score_body(h_ref, w1_ref, b1_ref, w2_ref, w_ref, er_ref):
    h = h_ref[...]
    t = jnp.tanh(
        jnp.dot(h, w1_ref[...], preferred_element_type=jnp.float32)
        + b1_ref[...]
    )
    w2 = w2_ref[...]
    s = lax.dot_general(
        t, w2, (((1,), (1,)), ((), ())),
        preferred_element_type=jnp.float32,
    )
    e = jnp.exp(s - jnp.sum(jnp.abs(w2)))
    w_ref[...] = e * h
    er_ref[...] = jnp.broadcast_to(e, (e.shape[0], 16))


def _scatter_body(w_hbm, e_hbm, idx_hbm, outp, outd,
                  bufs, zbuf, tbufp, dloc, tbufd, drow,
                  accp, accd):
    cid = lax.axis_index("c")
    sid = lax.axis_index("s")
    wid = sid * _NC + cid

    zero = jnp.zeros((16,), jnp.float32)
    for r in range(32):
        for g in range(8):
            zbuf[r, pl.ds(g * 16, 16)] = zero
    for g in range(2):
        dloc[pl.ds(g * 16, 16)] = zero

    def zrow(k, c):
        pltpu.sync_copy(zbuf, accp.at[pl.ds(sid * _G1 + k * 32, 32)])
        pltpu.sync_copy(dloc, accd.at[pl.ds(sid * _G1 + k * 32, 32)])
        return c
    lax.fori_loop(0, 16, zrow, 0)
    plsc.subcore_barrier()

    def _start(j):
        wb, eb, ib, sw, se, si = bufs[j % 2]
        base = (wid * _CH + j) * _RW
        return (pltpu.async_copy(w_hbm.at[pl.ds(base, _RW)], wb, sw),
                pltpu.async_copy(e_hbm.at[pl.ds(base, _RW)], eb, se),
                pltpu.async_copy(idx_hbm.at[pl.ds(base, _RW)], ib, si))

    descs = [None, None]
    descs[0] = _start(0)
    for j in range(_CH):
        if j + 1 < _CH:
            descs[(j + 1) % 2] = _start(j + 1)
        for dsc in descs[j % 2]:
            dsc.wait()
        wb, eb, ib = bufs[j % 2][:3]
        off = sid * _G1
        for g in range(8):
            ib[pl.ds(g * 16, 16)] = ib[pl.ds(g * 16, 16)] + off
        pltpu.sync_copy(wb, accp.at[ib], add=True)
        pltpu.sync_copy(eb, accd.at[ib], add=True)
    plsc.subcore_barrier()

    def pred(t, c):
        pltpu.sync_copy(accp.at[pl.ds(t * _G1 + sid * 32, 32)], tbufp)

        def prow(r, c2):
            for g in range(8):
                zbuf[r, pl.ds(g * 16, 16)] = (zbuf[r, pl.ds(g * 16, 16)]
                                              + tbufp[r, pl.ds(g * 16, 16)])
            return c2
        lax.fori_loop(0, 32, prow, 0)
        return c
    lax.fori_loop(0, 16, pred, 0)
    pltpu.sync_copy(zbuf, outp.at[cid, pl.ds(sid * 32, 32)])

    for g in range(2):
        dloc[pl.ds(g * 16, 16)] = zero

    def dred(t, c):
        pltpu.sync_copy(accd.at[pl.ds(t * _G1 + sid * 32, 32)], tbufd)
        for g in range(2):
            dloc[pl.ds(g * 16, 16)] = (dloc[pl.ds(g * 16, 16)]
                                       + tbufd[pl.ds(g * 16, 16)])
        return c
    lax.fori_loop(0, 16, dred, 0)
    for g2 in range(2):
        v = dloc[pl.ds(g2 * 16, 16)]
        for k in range(16):
            row = jnp.full((16,), v[k], jnp.float32)
            for g in range(8):
                drow[g2 * 16 + k, pl.ds(g * 16, 16)] = row
    pltpu.sync_copy(drow, outd.at[cid, pl.ds(sid * 32, 32)])


def _combine_body(p_ref, d_ref, o_ref):
    p = p_ref[...]
    d3 = d_ref[...]
    ps = p[0] + p[1]
    d = d3[0, :, 0:1] + d3[1, :, 0:1]
    o_ref[...] = ps / jnp.where(d > 0.0, d, 1.0)


def kernel(h, batch, W1, b1, W2, b2):
    del b2
    w2r = W2.reshape(1, _H)
    b1r = b1.reshape(1, _H)

    grid = (_N + _BLK - 1) // _BLK
    w, er = pl.pallas_call(
        _score_body,
        grid=(grid,),
        in_specs=[
            pl.BlockSpec((_BLK, _H), lambda i: (i, 0)),
            pl.BlockSpec((_H, _H), lambda i: (0, 0)),
            pl.BlockSpec((1, _H), lambda i: (0, 0)),
            pl.BlockSpec((1, _H), lambda i: (0, 0)),
        ],
        out_specs=[
            pl.BlockSpec((_BLK, _H), lambda i: (i, 0)),
            pl.BlockSpec((_BLK, 16), lambda i: (i, 0)),
        ],
        out_shape=[
            jax.ShapeDtypeStruct((_NP, _H), jnp.float32),
            jax.ShapeDtypeStruct((_NP, 16), jnp.float32),
        ],
    )(h, W1, b1r, w2r)

    pad_ids = jnp.full((_NP - _N,), _G, jnp.int32)
    idx_flat = jnp.concatenate([batch.astype(jnp.int32), pad_ids])

    scatter = pl.kernel(
        _scatter_body,
        out_type=[
            jax.ShapeDtypeStruct((_NC, _G, _H), jnp.float32),
            jax.ShapeDtypeStruct((_NC, _G, _H), jnp.float32),
        ],
        mesh=plsc.VectorSubcoreMesh(
            core_axis_name="c", subcore_axis_name="s",
            num_cores=_NC, num_subcores=_NS,
        ),
        scratch_types=[
            tuple(
                (pltpu.VMEM((_RW, _H), jnp.float32),
                 pltpu.VMEM((_RW,), jnp.float32),
                 pltpu.VMEM((_RW,), jnp.int32),
                 pltpu.SemaphoreType.DMA,
                 pltpu.SemaphoreType.DMA,
                 pltpu.SemaphoreType.DMA)
                for _ in range(2)
            ),
            pltpu.VMEM((32, _H), jnp.float32),
            pltpu.VMEM((32, _H), jnp.float32),
            pltpu.VMEM((32,), jnp.float32),
            pltpu.VMEM((32,), jnp.float32),
            pltpu.VMEM((32, _H), jnp.float32),
            pltpu.VMEM_SHARED((_NS * _G1, _H), jnp.float32),
            pltpu.VMEM_SHARED((_NS * _G1,), jnp.float32),
        ],
    )
    p, dd = scatter(w, er[:, 0], idx_flat)

    out = pl.pallas_call(
        _combine_body,
        out_shape=jax.ShapeDtypeStruct((_G, _H), jnp.float32),
    )(p, dd)
    return out
